# baseline (speedup 1.0000x reference)
import functools

import numpy as np
import jax
import jax.numpy as jnp
from jax.experimental import pallas as pl
from jax.experimental.pallas import tpu as pltpu

BATCH = 16384
ATOMS = 200
NUM_SPECIES = 8

BR = 2048

_VINV = np.linalg.inv(
    np.vander(np.arange(NUM_SPECIES), NUM_SPECIES, increasing=True)
    .astype(np.float64))


def _tc_body(tab_ref, spec_ref, sp_out_ref, out_ref):
    coef = [None] * NUM_SPECIES
    for m in range(NUM_SPECIES):
        c = None
        for k in range(NUM_SPECIES):
            w = float(_VINV[m, k])
            if w == 0.0:
                continue
            term = w * tab_ref[k]
            c = term if c is None else c + term
        coef[m] = c

    x = spec_ref[...]
    sp_out_ref[...] = x
    xf = x.astype(jnp.float32)
    val = jnp.full(xf.shape, 0.0, jnp.float32) + coef[NUM_SPECIES - 1]
    for m in range(NUM_SPECIES - 2, -1, -1):
        val = val * xf + coef[m]
    ones = jnp.ones((ATOMS, 8), jnp.float32)
    out_ref[...] = jax.lax.dot_general(
        val, ones, (((1,), (0,)), ((), ())),
        preferred_element_type=jnp.float32)


@functools.partial(jax.jit)
def _tc_shift(species, energies, self_energies):
    grid = (BATCH // BR,)
    sp_out, sae8 = pl.pallas_call(
        _tc_body,
        grid=grid,
        in_specs=[
            pl.BlockSpec(memory_space=pltpu.SMEM),
            pl.BlockSpec((BR, ATOMS), lambda i: (i, 0)),
        ],
        out_specs=[
            pl.BlockSpec((BR, ATOMS), lambda i: (i, 0)),
            pl.BlockSpec((BR, 8), lambda i: (i, 0)),
        ],
        out_shape=[
            jax.ShapeDtypeStruct((BATCH, ATOMS), jnp.int32),
            jax.ShapeDtypeStruct((BATCH, 8), jnp.float32),
        ],
        compiler_params=pltpu.CompilerParams(
            dimension_semantics=("arbitrary",)),
    )(self_energies, species)
    return sp_out, energies + sae8[:, 0]


def kernel(species, energies, self_energies):
    sp_out, shifted = _tc_shift(species, energies, self_energies)
    return (sp_out, shifted)


# speedup vs baseline: 1.2102x; 1.2102x over previous
import functools

import numpy as np
import jax
import jax.numpy as jnp
from jax.experimental import pallas as pl
from jax.experimental.pallas import tpu as pltpu

BATCH = 16384
ATOMS = 200
NUM_SPECIES = 8

CH = 2048
NBUF = 4
NCHUNK = BATCH // CH

_VINV = np.linalg.inv(
    np.vander(np.arange(NUM_SPECIES), NUM_SPECIES, increasing=True)
    .astype(np.float64))


def _tc_body(tab_ref, spec_hbm, out_ref, buf, sems):
    i = pl.program_id(0)

    def start(j):
        pltpu.make_async_copy(
            spec_hbm.at[pl.ds(j * CH, CH), :],
            buf.at[jax.lax.rem(j, NBUF)],
            sems.at[jax.lax.rem(j, NBUF)],
        ).start()

    @pl.when(i == 0)
    def _prologue():
        for j in range(min(NBUF - 1, NCHUNK)):
            start(j)

    @pl.when(i + NBUF - 1 < NCHUNK)
    def _ahead():
        start(i + NBUF - 1)

    pltpu.make_async_copy(
        spec_hbm.at[pl.ds(i * CH, CH), :],
        buf.at[jax.lax.rem(i, NBUF)],
        sems.at[jax.lax.rem(i, NBUF)],
    ).wait()

    coef = [None] * NUM_SPECIES
    for m in range(NUM_SPECIES):
        c = None
        for k in range(NUM_SPECIES):
            w = float(_VINV[m, k])
            if w == 0.0:
                continue
            term = w * tab_ref[k]
            c = term if c is None else c + term
        coef[m] = c

    xf = buf[jax.lax.rem(i, NBUF)].astype(jnp.float32)
    val = jnp.full(xf.shape, 0.0, jnp.float32) + coef[NUM_SPECIES - 1]
    for m in range(NUM_SPECIES - 2, -1, -1):
        val = val * xf + coef[m]
    ones = jnp.ones((ATOMS, 8), jnp.float32)
    out_ref[...] = jax.lax.dot_general(
        val, ones, (((1,), (0,)), ((), ())),
        preferred_element_type=jnp.float32)


@functools.partial(jax.jit)
def _tc_shift(species, energies, self_energies):
    sae8 = pl.pallas_call(
        _tc_body,
        grid=(NCHUNK,),
        in_specs=[
            pl.BlockSpec(memory_space=pltpu.SMEM),
            pl.BlockSpec(memory_space=pl.ANY),
        ],
        out_specs=pl.BlockSpec((CH, 8), lambda i: (i, 0)),
        out_shape=jax.ShapeDtypeStruct((BATCH, 8), jnp.float32),
        scratch_shapes=[
            pltpu.VMEM((NBUF, CH, ATOMS), jnp.int32),
            pltpu.SemaphoreType.DMA((NBUF,)),
        ],
        compiler_params=pltpu.CompilerParams(
            dimension_semantics=("arbitrary",)),
    )(self_energies, species)
    return energies + sae8[:, 0]


def kernel(species, energies, self_energies):
    shifted = _tc_shift(species, energies, self_energies)
    return (species, shifted)
